# baseline (speedup 1.0000x reference)
import functools

import jax
import jax.numpy as jnp
from jax import lax
from jax.experimental import pallas as pl
from jax.experimental.pallas import tpu as pltpu
from jax.experimental.pallas import tpu_sc as plsc

_LANES = 16
_NSLOT = 4
_CH = (104, 96)


def _make_sc_add(batch, maxlen, dim):
    info = plsc.get_sparse_core_info()
    nc, ns = info.num_cores, info.num_subcores
    nw = nc * ns
    assert batch % nw == 0 and dim % _LANES == 0 and sum(_CH) == maxlen
    b_per_w = batch // nw
    sp_rows = ns * _NSLOT * maxlen

    mesh = plsc.VectorSubcoreMesh(core_axis_name="c", subcore_axis_name="s")

    @functools.partial(
        pl.kernel,
        out_type=jax.ShapeDtypeStruct((batch, maxlen, dim), jnp.float32),
        mesh=mesh,
        scratch_types=[
            pltpu.VMEM((maxlen, dim), jnp.float32),
            pltpu.VMEM((_NSLOT, _CH[0]), jnp.int32),
            pltpu.VMEM((_NSLOT, _CH[1]), jnp.int32),
            pltpu.VMEM_SHARED((sp_rows, dim), jnp.float32),
            pltpu.SemaphoreType.DMA,
            pltpu.SemaphoreType.DMA,
            pltpu.SemaphoreType.DMA,
            pltpu.SemaphoreType.DMA,
            pltpu.SemaphoreType.DMA,
            pltpu.SemaphoreType.DMA,
            pltpu.SemaphoreType.DMA,
            pltpu.SemaphoreType.DMA,
            pltpu.SemaphoreType.DMA,
            pltpu.SemaphoreType.DMA,
            pltpu.SemaphoreType.DMA,
            pltpu.SemaphoreType.DMA,
        ],
    )
    def sc_add(x_hbm, pos_hbm, out_hbm, pos_v, idxa, idxb, sp,
               in0, in1, in2, in3, ad0, ad1, ad2, ad3, ot0, ot1, ot2, ot3):
        cid = lax.axis_index("c")
        sid = lax.axis_index("s")
        wid = sid * nc + cid
        base = wid * b_per_w
        tile_base = sid * (_NSLOT * maxlen)
        insems = [in0, in1, in2, in3]
        addsems = [ad0, ad1, ad2, ad3]
        outsems = [ot0, ot1, ot2, ot3]

        pltpu.sync_copy(pos_hbm, pos_v)

        iot = lax.iota(jnp.int32, _LANES)
        for s in range(_NSLOT):
            rowbase = tile_base + s * maxlen
            for o in (0, 16, 32, 48, 64, 80, _CH[0] - _LANES):
                idxa[s, pl.ds(o, _LANES)] = rowbase + o + iot
            for o in (0, 16, 32, 48, 64, _CH[1] - _LANES):
                idxb[s, pl.ds(o, _LANES)] = rowbase + _CH[0] + o + iot

        def sp_slot(s):
            return sp.at[pl.ds(tile_base + s * maxlen, maxlen)]

        def start_in(r, s):
            pltpu.async_copy(x_hbm.at[base + r], sp_slot(s), insems[s])

        def wait_in(r, s):
            pltpu.make_async_copy(
                x_hbm.at[base + r], sp_slot(s), insems[s]).wait()

        def start_adds(s):
            pltpu.async_copy(pos_v.at[pl.ds(0, _CH[0])],
                             sp.at[idxa.at[s]], addsems[s], add=True)
            pltpu.async_copy(pos_v.at[pl.ds(_CH[0], _CH[1])],
                             sp.at[idxb.at[s]], addsems[s], add=True)

        def wait_adds(s):
            pltpu.make_async_copy(pos_v.at[pl.ds(0, _CH[0])],
                                  sp.at[idxa.at[s]], addsems[s]).wait()
            pltpu.make_async_copy(pos_v.at[pl.ds(_CH[0], _CH[1])],
                                  sp.at[idxb.at[s]], addsems[s]).wait()

        def start_out(r, s):
            pltpu.async_copy(sp_slot(s), out_hbm.at[base + r], outsems[s])

        def wait_out(r, s):
            pltpu.make_async_copy(
                sp_slot(s), out_hbm.at[base + r], outsems[s]).wait()

        for s in range(_NSLOT):
            start_in(s, s)
        for r in range(b_per_w):
            s = r % _NSLOT
            wait_in(r, s)
            start_adds(s)
            if r >= 1:
                q = (r - 1) % _NSLOT
                wait_adds(q)
                start_out(r - 1, q)
            if 3 <= r < b_per_w - 1:
                t = (r + 1) % _NSLOT
                wait_out(r - 3, t)
                start_in(r + 1, t)
        last = b_per_w - 1
        wait_adds(last % _NSLOT)
        start_out(last, last % _NSLOT)
        for rr in range(last - 3, last + 1):
            wait_out(rr, rr % _NSLOT)

    return sc_add


def kernel(x, pos_emb):
    batch, maxlen, dim = x.shape
    return _make_sc_add(batch, maxlen, dim)(x, pos_emb)


# speedup vs baseline: 1.0039x; 1.0039x over previous
import functools

import jax
import jax.numpy as jnp
from jax import lax
from jax.experimental import pallas as pl
from jax.experimental.pallas import tpu as pltpu
from jax.experimental.pallas import tpu_sc as plsc

_LANES = 16
_NSLOT = 3
_CH = (104, 96)


def _make_sc_add(batch, maxlen, dim):
    info = plsc.get_sparse_core_info()
    nc, ns = info.num_cores, info.num_subcores
    nw = nc * ns
    assert batch % nw == 0 and dim % _LANES == 0 and sum(_CH) == maxlen
    b_per_w = batch // nw
    sp_rows = ns * _NSLOT * maxlen

    mesh = plsc.VectorSubcoreMesh(core_axis_name="c", subcore_axis_name="s")

    @functools.partial(
        pl.kernel,
        out_type=jax.ShapeDtypeStruct((batch, maxlen, dim), jnp.float32),
        mesh=mesh,
        scratch_types=[
            pltpu.VMEM((maxlen, dim), jnp.float32),
            pltpu.VMEM((_NSLOT, _CH[0]), jnp.int32),
            pltpu.VMEM((_NSLOT, _CH[1]), jnp.int32),
            pltpu.VMEM_SHARED((sp_rows, dim), jnp.float32),
            pltpu.SemaphoreType.DMA,
            pltpu.SemaphoreType.DMA,
            pltpu.SemaphoreType.DMA,
            pltpu.SemaphoreType.DMA,
            pltpu.SemaphoreType.DMA,
            pltpu.SemaphoreType.DMA,
            pltpu.SemaphoreType.DMA,
            pltpu.SemaphoreType.DMA,
            pltpu.SemaphoreType.DMA,
        ],
    )
    def sc_add(x_hbm, pos_hbm, out_hbm, pos_v, idxa, idxb, sp,
               in0, in1, in2, ad0, ad1, ad2, ot0, ot1, ot2):
        cid = lax.axis_index("c")
        sid = lax.axis_index("s")
        wid = sid * nc + cid
        base = wid * b_per_w
        tile_base = sid * (_NSLOT * maxlen)
        insems, addsems, outsems = [in0, in1, in2], [ad0, ad1, ad2], [ot0, ot1, ot2]

        pltpu.sync_copy(pos_hbm, pos_v)

        iot = lax.iota(jnp.int32, _LANES)
        for s in range(_NSLOT):
            rowbase = tile_base + s * maxlen
            for o in (0, 16, 32, 48, 64, 80, _CH[0] - _LANES):
                idxa[s, pl.ds(o, _LANES)] = rowbase + o + iot
            for o in (0, 16, 32, 48, 64, _CH[1] - _LANES):
                idxb[s, pl.ds(o, _LANES)] = rowbase + _CH[0] + o + iot

        def sp_slot(s):
            return sp.at[pl.ds(tile_base + s * maxlen, maxlen)]

        def start_in(r, s):
            pltpu.async_copy(x_hbm.at[base + r], sp_slot(s), insems[s])

        def wait_in(r, s):
            pltpu.make_async_copy(
                x_hbm.at[base + r], sp_slot(s), insems[s]).wait()

        def start_adds(s):
            pltpu.async_copy(pos_v.at[pl.ds(0, _CH[0])],
                             sp.at[idxa.at[s]], addsems[s], add=True)
            pltpu.async_copy(pos_v.at[pl.ds(_CH[0], _CH[1])],
                             sp.at[idxb.at[s]], addsems[s], add=True)

        def wait_adds(s):
            pltpu.make_async_copy(pos_v.at[pl.ds(0, _CH[0])],
                                  sp.at[idxa.at[s]], addsems[s]).wait()
            pltpu.make_async_copy(pos_v.at[pl.ds(_CH[0], _CH[1])],
                                  sp.at[idxb.at[s]], addsems[s]).wait()

        def start_out(r, s):
            pltpu.async_copy(sp_slot(s), out_hbm.at[base + r], outsems[s])

        def wait_out(r, s):
            pltpu.make_async_copy(
                sp_slot(s), out_hbm.at[base + r], outsems[s]).wait()

        for s in range(_NSLOT):
            start_in(s, s)
        for r in range(b_per_w):
            s = r % _NSLOT
            wait_in(r, s)
            start_adds(s)
            if r >= 1:
                q = (r - 1) % _NSLOT
                wait_adds(q)
                start_out(r - 1, q)
            if 2 <= r < b_per_w - 1:
                t = (r + 1) % _NSLOT
                wait_out(r - 2, t)
                start_in(r + 1, t)
        last = b_per_w - 1
        wait_adds(last % _NSLOT)
        start_out(last, last % _NSLOT)
        wait_out(last - 1, (last - 1) % _NSLOT)
        wait_out(last, last % _NSLOT)

    return sc_add


def kernel(x, pos_emb):
    batch, maxlen, dim = x.shape
    return _make_sc_add(batch, maxlen, dim)(x, pos_emb)
